# baseline (speedup 1.0000x reference)
import functools

import jax
import jax.numpy as jnp
from jax import lax
from jax.experimental import pallas as pl
from jax.experimental.pallas import tpu as pltpu
from jax.experimental.pallas import tpu_sc as plsc

NC, NS, L = 2, 16, 16
NW = NC * NS
CH = 64
SPLIT0_PCT = 58

_SC_PARAMS = pltpu.CompilerParams(needs_layout_passes=False)


def _mesh():
    return plsc.VectorSubcoreMesh(core_axis_name="c", subcore_axis_name="s")


@functools.partial(jax.jit, static_argnums=(2, 3))
def _dis_call(colp, wp, EP, NP):
    EPW = EP // NS
    S = NP // NS

    @functools.partial(
        pl.kernel,
        out_type=jax.ShapeDtypeStruct((NP,), jnp.float32),
        mesh=_mesh(),
        compiler_params=_SC_PARAMS,
        scratch_types=[
            pltpu.VMEM_SHARED((NS, NP), jnp.float32),
            pltpu.VMEM((EPW,), jnp.int32),
            pltpu.VMEM((EPW,), jnp.float32),
            pltpu.VMEM((NP,), jnp.float32),
            pltpu.VMEM((S,), jnp.float32),
            pltpu.VMEM((S,), jnp.float32),
        ],
    )
    def dis_kernel(col_hbm, w_hbm, dis_hbm, part_sh, col_v, w_v, deg_v, racc_v, rtmp_v):
        cid = lax.axis_index("c")
        sid = lax.axis_index("s")

        def z(i, _):
            deg_v[pl.ds(i * L, L)] = jnp.zeros((L,), jnp.float32)
            return 0

        lax.fori_loop(0, NP // L, z, 0)
        base = sid * EPW
        pltpu.sync_copy(col_hbm.at[pl.ds(base, EPW)], col_v)
        pltpu.sync_copy(w_hbm.at[pl.ds(base, EPW)], w_v)

        def body(i, _):
            idx = col_v[pl.ds(i * L, L)]
            wv = w_v[pl.ds(i * L, L)]
            plsc.addupdate_scatter(deg_v, [idx], wv)
            return 0

        lax.fori_loop(0, EPW // L, body, 0)
        pltpu.sync_copy(deg_v, part_sh.at[sid])
        plsc.subcore_barrier()

        def z2(i, _):
            racc_v[pl.ds(i * L, L)] = jnp.ones((L,), jnp.float32)
            return 0

        lax.fori_loop(0, S // L, z2, 0)

        def red(t, _):
            pltpu.sync_copy(part_sh.at[t, pl.ds(sid * S, S)], rtmp_v)

            def addv(i, _):
                racc_v[pl.ds(i * L, L)] = (
                    racc_v[pl.ds(i * L, L)] + rtmp_v[pl.ds(i * L, L)]
                )
                return 0

            lax.fori_loop(0, S // L, addv, 0)
            return 0

        lax.fori_loop(0, NS, red, 0)

        def nrs(i, _):
            xx = racc_v[pl.ds(i * L, L)]
            ii = plsc.bitcast(xx, jnp.int32)
            ii = 0x5F3759DF - (ii >> 1)
            yy = plsc.bitcast(ii, jnp.float32)
            half = xx * 0.5
            yy = yy * (1.5 - half * yy * yy)
            yy = yy * (1.5 - half * yy * yy)
            yy = yy * (1.5 - half * yy * yy)
            racc_v[pl.ds(i * L, L)] = yy
            return 0

        lax.fori_loop(0, S // L, nrs, 0)

        @pl.when(cid == 0)
        def _():
            pltpu.sync_copy(racc_v, dis_hbm.at[pl.ds(sid * S, S)])

    return dis_kernel(colp, wp)


def _y_call(xp, W, dis_col, R):
    NP = xp.shape[0]
    G = NP // R
    D = W.shape[1]

    def body(x_ref, w_ref, dis_ref, y_ref, sinv_ref):
        @pl.when(pl.program_id(0) == 0)
        def _():
            Wm = w_ref[...]
            A = lax.dot_general(
                Wm, Wm, (((1,), (1,)), ((), ())), preferred_element_type=jnp.float32
            )
            r_i = lax.broadcasted_iota(jnp.int32, (8, D), 0).astype(jnp.float32)
            c_i = lax.broadcasted_iota(jnp.int32, (8, D), 1).astype(jnp.float32)
            t = (r_i * 128.0 + c_i) * 0.6180339887
            v0 = t - jnp.floor(t) - 0.5

            def it(_, v):
                v = lax.dot_general(
                    v, A, (((1,), (0,)), ((), ())),
                    preferred_element_type=jnp.float32,
                )
                nrm = lax.rsqrt(jnp.sum(v * v, axis=1, keepdims=True) + 1e-30)
                return v * nrm

            v = lax.fori_loop(0, 80, it, v0)
            av = lax.dot_general(
                v, A, (((1,), (0,)), ((), ())), preferred_element_type=jnp.float32
            )
            rq = jnp.sum(av * v, axis=1, keepdims=True)
            lam = jnp.max(rq)
            sinv_ref[0, 0] = lax.rsqrt(lam)

        si = sinv_ref[0, 0]
        xw = lax.dot_general(
            x_ref[...], w_ref[...], (((1,), (1,)), ((), ())),
            preferred_element_type=jnp.float32,
        )
        y_ref[...] = xw * si * dis_ref[...]

    return pl.pallas_call(
        body,
        grid=(G,),
        in_specs=[
            pl.BlockSpec((R, D), lambda i: (i, 0)),
            pl.BlockSpec((D, D), lambda i: (0, 0)),
            pl.BlockSpec((R, 1), lambda i: (i, 0)),
        ],
        out_specs=pl.BlockSpec((R, D), lambda i: (i, 0)),
        out_shape=jax.ShapeDtypeStruct((NP, D), jnp.float32),
        scratch_shapes=[pltpu.SMEM((1, 1), jnp.float32)],
    )(xp, W, dis_col)


@functools.partial(jax.jit, static_argnums=(4, 5))
def _scatter_call(y, rowp, colp, wp, EP, NP):
    D = y.shape[1]
    NCHK_ALL = EP // CH // NS
    K0 = (NCHK_ALL * SPLIT0_PCT // 100) // 4 * 4
    K1 = NCHK_ALL - K0
    S = NP // NS
    NB = 4

    @functools.partial(
        pl.kernel,
        out_type=jax.ShapeDtypeStruct((NC, NP, D), jnp.float32),
        mesh=_mesh(),
        compiler_params=_SC_PARAMS,
        scratch_types=[
            pltpu.VMEM_SHARED((NP, D), jnp.float32),
            pltpu.VMEM((NB, CH), jnp.int32),
            pltpu.VMEM((NB, CH), jnp.int32),
            pltpu.VMEM((NB, CH), jnp.float32),
            pltpu.VMEM((NB, CH, D), jnp.float32),
            pltpu.VMEM((32, D), jnp.float32),
            pltpu.SemaphoreType.DMA,
            pltpu.SemaphoreType.DMA,
            pltpu.SemaphoreType.DMA,
            pltpu.SemaphoreType.DMA,
            pltpu.SemaphoreType.DMA,
            pltpu.SemaphoreType.DMA,
            pltpu.SemaphoreType.DMA,
            pltpu.SemaphoreType.DMA,
        ],
    )
    def scat(y_hbm, row_hbm, col_hbm, w_hbm, out_hbm,
             acc_sh, ri_v, ci_v, w_v, rows_v, zb_v,
             is0, is1, is2, is3, gs0, gs1, gs2, gs3):
        cid = lax.axis_index("c")
        sid = lax.axis_index("s")
        nchk = jnp.where(cid == 0, K0, K1)
        base_chunk = jnp.where(cid == 0, sid * K0, NS * K0 + sid * K1)
        base = base_chunk * CH
        isem = (is0, is1, is2, is3)
        gsem = (gs0, gs1, gs2, gs3)

        def start_idx(c, b):
            off = base + c * CH
            pltpu.async_copy(row_hbm.at[pl.ds(off, CH)], ri_v.at[b], isem[b])
            pltpu.async_copy(col_hbm.at[pl.ds(off, CH)], ci_v.at[b], isem[b])
            pltpu.async_copy(w_hbm.at[pl.ds(off, CH)], w_v.at[b], isem[b])

        def wait_idx(c, b):
            off = base + c * CH
            pltpu.make_async_copy(row_hbm.at[pl.ds(off, CH)], ri_v.at[b], isem[b]).wait()
            pltpu.make_async_copy(col_hbm.at[pl.ds(off, CH)], ci_v.at[b], isem[b]).wait()
            pltpu.make_async_copy(w_hbm.at[pl.ds(off, CH)], w_v.at[b], isem[b]).wait()

        def start_g(b):
            pltpu.async_copy(y_hbm.at[ri_v.at[b]], rows_v.at[b], gsem[b])

        def wait_g(b):
            pltpu.make_async_copy(y_hbm.at[ri_v.at[b]], rows_v.at[b], gsem[b]).wait()

        def process(b):
            def scale(r, _):
                wr = plsc.load_gather(w_v.at[b], [jnp.full((L,), r, jnp.int32)])
                for j in range(D // L):
                    rows_v[b, r, pl.ds(j * L, L)] = rows_v[b, r, pl.ds(j * L, L)] * wr
                return 0

            lax.fori_loop(0, CH, scale, 0)
            pltpu.sync_copy(rows_v.at[b], acc_sh.at[ci_v.at[b]], add=True)

        for b in range(NB):
            start_idx(b, b)

        def zr(i, _):
            for j in range(D // L):
                zb_v[i, pl.ds(j * L, L)] = jnp.zeros((L,), jnp.float32)
            return 0

        lax.fori_loop(0, 32, zr, 0)

        def zs(i, _):
            pltpu.sync_copy(zb_v, acc_sh.at[pl.ds(sid * S + i * 32, 32)])
            return 0

        lax.fori_loop(0, S // 32, zs, 0)

        wait_idx(0, 0)
        start_g(0)
        wait_idx(1, 1)
        start_g(1)
        plsc.subcore_barrier()

        def body(g, _):
            c0 = 4 * g
            for b in range(NB):
                cc = c0 + b

                @pl.when(cc + 2 < nchk)
                def _():
                    wait_idx(cc + 2, (b + 2) % NB)
                    start_g((b + 2) % NB)

                wait_g(b)
                process(b)

                @pl.when(cc + 4 < nchk)
                def _():
                    start_idx(cc + 4, b)

            return 0

        lax.fori_loop(0, nchk // 4, body, 0)
        plsc.subcore_barrier()
        pltpu.sync_copy(acc_sh.at[pl.ds(sid * S, S)], out_hbm.at[cid, pl.ds(sid * S, S)])

    return scat(y, rowp, colp, wp)


def _final_call(accp, y, dis_col, b2, R):
    NP, D = y.shape
    G = NP // R

    def body(a_ref, y_ref, dis_ref, b_ref, o_ref):
        acc = a_ref[0] + a_ref[1] + y_ref[...]
        o_ref[...] = acc * dis_ref[...] + b_ref[...]

    return pl.pallas_call(
        body,
        grid=(G,),
        in_specs=[
            pl.BlockSpec((NC, R, D), lambda i: (0, i, 0)),
            pl.BlockSpec((R, D), lambda i: (i, 0)),
            pl.BlockSpec((R, 1), lambda i: (i, 0)),
            pl.BlockSpec((1, D), lambda i: (0, 0)),
        ],
        out_specs=pl.BlockSpec((R, D), lambda i: (i, 0)),
        out_shape=jax.ShapeDtypeStruct((NP, D), jnp.float32),
    )(accp, y, dis_col, b2)


def kernel(x, edge_index, edge_weight, W, b):
    N, _ = x.shape
    E = edge_weight.shape[0]
    D = W.shape[1]

    NP = ((N + 1 + 2047) // 2048) * 2048
    EP = ((E + 4 * NW * CH - 1) // (4 * NW * CH)) * (4 * NW * CH)
    R = 2048

    row = edge_index[0]
    col = edge_index[1]
    pad_e = EP - E
    rowp = jnp.concatenate([row, jnp.zeros((pad_e,), jnp.int32)])
    colp = jnp.concatenate([col, jnp.full((pad_e,), N, jnp.int32)])
    wp = jnp.concatenate([edge_weight, jnp.zeros((pad_e,), jnp.float32)])
    xp = jnp.concatenate([x, jnp.zeros((NP - N, D), jnp.float32)], axis=0)

    dis = _dis_call(colp, wp, EP, NP)
    dis_col = dis[:, None]
    y = _y_call(xp, W, dis_col, R)
    accp = _scatter_call(y, rowp, colp, wp, EP, NP)
    out = _final_call(accp, y, dis_col, b[None, :], R)
    return out[:N]


# speedup vs baseline: 1.0358x; 1.0358x over previous
import functools

import jax
import jax.numpy as jnp
from jax import lax
from jax.experimental import pallas as pl
from jax.experimental.pallas import tpu as pltpu
from jax.experimental.pallas import tpu_sc as plsc

NC, NS, L = 2, 16, 16
NW = NC * NS
CH = 64
SPLIT0_PCT = 70

_SC_PARAMS = pltpu.CompilerParams(needs_layout_passes=False)


def _mesh():
    return plsc.VectorSubcoreMesh(core_axis_name="c", subcore_axis_name="s")


@functools.partial(jax.jit, static_argnums=(2, 3))
def _dis_call(colp, wp, EP, NP):
    EPW = EP // NS
    S = NP // NS

    @functools.partial(
        pl.kernel,
        out_type=jax.ShapeDtypeStruct((NP,), jnp.float32),
        mesh=_mesh(),
        compiler_params=_SC_PARAMS,
        scratch_types=[
            pltpu.VMEM_SHARED((NS, NP), jnp.float32),
            pltpu.VMEM((EPW,), jnp.int32),
            pltpu.VMEM((EPW,), jnp.float32),
            pltpu.VMEM((NP,), jnp.float32),
            pltpu.VMEM((S,), jnp.float32),
            pltpu.VMEM((S,), jnp.float32),
        ],
    )
    def dis_kernel(col_hbm, w_hbm, dis_hbm, part_sh, col_v, w_v, deg_v, racc_v, rtmp_v):
        cid = lax.axis_index("c")
        sid = lax.axis_index("s")

        def z(i, _):
            deg_v[pl.ds(i * L, L)] = jnp.zeros((L,), jnp.float32)
            return 0

        lax.fori_loop(0, NP // L, z, 0)
        base = sid * EPW
        pltpu.sync_copy(col_hbm.at[pl.ds(base, EPW)], col_v)
        pltpu.sync_copy(w_hbm.at[pl.ds(base, EPW)], w_v)

        def body(i, _):
            idx = col_v[pl.ds(i * L, L)]
            wv = w_v[pl.ds(i * L, L)]
            plsc.addupdate_scatter(deg_v, [idx], wv)
            return 0

        lax.fori_loop(0, EPW // L, body, 0)
        pltpu.sync_copy(deg_v, part_sh.at[sid])
        plsc.subcore_barrier()

        def z2(i, _):
            racc_v[pl.ds(i * L, L)] = jnp.ones((L,), jnp.float32)
            return 0

        lax.fori_loop(0, S // L, z2, 0)

        def red(t, _):
            pltpu.sync_copy(part_sh.at[t, pl.ds(sid * S, S)], rtmp_v)

            def addv(i, _):
                racc_v[pl.ds(i * L, L)] = (
                    racc_v[pl.ds(i * L, L)] + rtmp_v[pl.ds(i * L, L)]
                )
                return 0

            lax.fori_loop(0, S // L, addv, 0)
            return 0

        lax.fori_loop(0, NS, red, 0)

        def nrs(i, _):
            xx = racc_v[pl.ds(i * L, L)]
            ii = plsc.bitcast(xx, jnp.int32)
            ii = 0x5F3759DF - (ii >> 1)
            yy = plsc.bitcast(ii, jnp.float32)
            half = xx * 0.5
            yy = yy * (1.5 - half * yy * yy)
            yy = yy * (1.5 - half * yy * yy)
            yy = yy * (1.5 - half * yy * yy)
            racc_v[pl.ds(i * L, L)] = yy
            return 0

        lax.fori_loop(0, S // L, nrs, 0)

        @pl.when(cid == 0)
        def _():
            pltpu.sync_copy(racc_v, dis_hbm.at[pl.ds(sid * S, S)])

    return dis_kernel(colp, wp)


def _y_call(xp, W, dis_col, R):
    NP = xp.shape[0]
    G = NP // R
    D = W.shape[1]

    def body(x_ref, w_ref, dis_ref, y_ref, sinv_ref):
        @pl.when(pl.program_id(0) == 0)
        def _():
            Wm = w_ref[...]
            A = lax.dot_general(
                Wm, Wm, (((1,), (1,)), ((), ())), preferred_element_type=jnp.float32
            )
            r_i = lax.broadcasted_iota(jnp.int32, (8, D), 0).astype(jnp.float32)
            c_i = lax.broadcasted_iota(jnp.int32, (8, D), 1).astype(jnp.float32)
            t = (r_i * 128.0 + c_i) * 0.6180339887
            v0 = t - jnp.floor(t) - 0.5

            def it(_, v):
                v = lax.dot_general(
                    v, A, (((1,), (0,)), ((), ())),
                    preferred_element_type=jnp.float32,
                )
                nrm = lax.rsqrt(jnp.sum(v * v, axis=1, keepdims=True) + 1e-30)
                return v * nrm

            v = lax.fori_loop(0, 80, it, v0)
            av = lax.dot_general(
                v, A, (((1,), (0,)), ((), ())), preferred_element_type=jnp.float32
            )
            rq = jnp.sum(av * v, axis=1, keepdims=True)
            lam = jnp.max(rq)
            sinv_ref[0, 0] = lax.rsqrt(lam)

        si = sinv_ref[0, 0]
        xw = lax.dot_general(
            x_ref[...], w_ref[...], (((1,), (1,)), ((), ())),
            preferred_element_type=jnp.float32,
        )
        y_ref[...] = xw * si * dis_ref[...]

    return pl.pallas_call(
        body,
        grid=(G,),
        in_specs=[
            pl.BlockSpec((R, D), lambda i: (i, 0)),
            pl.BlockSpec((D, D), lambda i: (0, 0)),
            pl.BlockSpec((R, 1), lambda i: (i, 0)),
        ],
        out_specs=pl.BlockSpec((R, D), lambda i: (i, 0)),
        out_shape=jax.ShapeDtypeStruct((NP, D), jnp.float32),
        scratch_shapes=[pltpu.SMEM((1, 1), jnp.float32)],
    )(xp, W, dis_col)


@functools.partial(jax.jit, static_argnums=(4, 5))
def _scatter_call(y, rowp, colp, wp, EP, NP):
    D = y.shape[1]
    NCHK_ALL = EP // CH // NS
    K0 = (NCHK_ALL * SPLIT0_PCT // 100) // 4 * 4
    K1 = NCHK_ALL - K0
    S = NP // NS
    NB = 4

    @functools.partial(
        pl.kernel,
        out_type=jax.ShapeDtypeStruct((NC, NP, D), jnp.float32),
        mesh=_mesh(),
        compiler_params=_SC_PARAMS,
        scratch_types=[
            pltpu.VMEM_SHARED((NP, D), jnp.float32),
            pltpu.VMEM((NB, CH), jnp.int32),
            pltpu.VMEM((NB, CH), jnp.int32),
            pltpu.VMEM((NB, CH), jnp.float32),
            pltpu.VMEM((NB, CH, D), jnp.float32),
            pltpu.VMEM((32, D), jnp.float32),
            pltpu.SemaphoreType.DMA,
            pltpu.SemaphoreType.DMA,
            pltpu.SemaphoreType.DMA,
            pltpu.SemaphoreType.DMA,
            pltpu.SemaphoreType.DMA,
            pltpu.SemaphoreType.DMA,
            pltpu.SemaphoreType.DMA,
            pltpu.SemaphoreType.DMA,
        ],
    )
    def scat(y_hbm, row_hbm, col_hbm, w_hbm, out_hbm,
             acc_sh, ri_v, ci_v, w_v, rows_v, zb_v,
             is0, is1, is2, is3, gs0, gs1, gs2, gs3):
        cid = lax.axis_index("c")
        sid = lax.axis_index("s")
        nchk = jnp.where(cid == 0, K0, K1)
        base_chunk = jnp.where(cid == 0, sid * K0, NS * K0 + sid * K1)
        base = base_chunk * CH
        isem = (is0, is1, is2, is3)
        gsem = (gs0, gs1, gs2, gs3)

        def start_idx(c, b):
            off = base + c * CH
            pltpu.async_copy(row_hbm.at[pl.ds(off, CH)], ri_v.at[b], isem[b])
            pltpu.async_copy(col_hbm.at[pl.ds(off, CH)], ci_v.at[b], isem[b])
            pltpu.async_copy(w_hbm.at[pl.ds(off, CH)], w_v.at[b], isem[b])

        def wait_idx(c, b):
            off = base + c * CH
            pltpu.make_async_copy(row_hbm.at[pl.ds(off, CH)], ri_v.at[b], isem[b]).wait()
            pltpu.make_async_copy(col_hbm.at[pl.ds(off, CH)], ci_v.at[b], isem[b]).wait()
            pltpu.make_async_copy(w_hbm.at[pl.ds(off, CH)], w_v.at[b], isem[b]).wait()

        def start_g(b):
            pltpu.async_copy(y_hbm.at[ri_v.at[b]], rows_v.at[b], gsem[b])

        def wait_g(b):
            pltpu.make_async_copy(y_hbm.at[ri_v.at[b]], rows_v.at[b], gsem[b]).wait()

        def process(b):
            def scale(r, _):
                wr = plsc.load_gather(w_v.at[b], [jnp.full((L,), r, jnp.int32)])
                for j in range(D // L):
                    rows_v[b, r, pl.ds(j * L, L)] = rows_v[b, r, pl.ds(j * L, L)] * wr
                return 0

            lax.fori_loop(0, CH, scale, 0)
            pltpu.sync_copy(rows_v.at[b], acc_sh.at[ci_v.at[b]], add=True)

        for b in range(NB):
            start_idx(b, b)

        def zr(i, _):
            for j in range(D // L):
                zb_v[i, pl.ds(j * L, L)] = jnp.zeros((L,), jnp.float32)
            return 0

        lax.fori_loop(0, 32, zr, 0)

        def zs(i, _):
            pltpu.sync_copy(zb_v, acc_sh.at[pl.ds(sid * S + i * 32, 32)])
            return 0

        lax.fori_loop(0, S // 32, zs, 0)

        wait_idx(0, 0)
        start_g(0)
        wait_idx(1, 1)
        start_g(1)
        plsc.subcore_barrier()

        def body(g, _):
            c0 = 4 * g
            for b in range(NB):
                cc = c0 + b

                @pl.when(cc + 2 < nchk)
                def _():
                    wait_idx(cc + 2, (b + 2) % NB)
                    start_g((b + 2) % NB)

                wait_g(b)
                process(b)

                @pl.when(cc + 4 < nchk)
                def _():
                    start_idx(cc + 4, b)

            return 0

        lax.fori_loop(0, nchk // 4, body, 0)
        plsc.subcore_barrier()
        pltpu.sync_copy(acc_sh.at[pl.ds(sid * S, S)], out_hbm.at[cid, pl.ds(sid * S, S)])

    return scat(y, rowp, colp, wp)


def _final_call(accp, y, dis_col, b2, R):
    NP, D = y.shape
    G = NP // R

    def body(a_ref, y_ref, dis_ref, b_ref, o_ref):
        acc = a_ref[0] + a_ref[1] + y_ref[...]
        o_ref[...] = acc * dis_ref[...] + b_ref[...]

    return pl.pallas_call(
        body,
        grid=(G,),
        in_specs=[
            pl.BlockSpec((NC, R, D), lambda i: (0, i, 0)),
            pl.BlockSpec((R, D), lambda i: (i, 0)),
            pl.BlockSpec((R, 1), lambda i: (i, 0)),
            pl.BlockSpec((1, D), lambda i: (0, 0)),
        ],
        out_specs=pl.BlockSpec((R, D), lambda i: (i, 0)),
        out_shape=jax.ShapeDtypeStruct((NP, D), jnp.float32),
    )(accp, y, dis_col, b2)


def kernel(x, edge_index, edge_weight, W, b):
    N, _ = x.shape
    E = edge_weight.shape[0]
    D = W.shape[1]

    NP = ((N + 1 + 2047) // 2048) * 2048
    EP = ((E + 4 * NW * CH - 1) // (4 * NW * CH)) * (4 * NW * CH)
    R = 2048

    row = edge_index[0]
    col = edge_index[1]
    pad_e = EP - E
    rowp = jnp.concatenate([row, jnp.zeros((pad_e,), jnp.int32)])
    colp = jnp.concatenate([col, jnp.full((pad_e,), N, jnp.int32)])
    wp = jnp.concatenate([edge_weight, jnp.zeros((pad_e,), jnp.float32)])
    xp = jnp.concatenate([x, jnp.zeros((NP - N, D), jnp.float32)], axis=0)

    dis = _dis_call(colp, wp, EP, NP)
    dis_col = dis[:, None]
    y = _y_call(xp, W, dis_col, R)
    accp = _scatter_call(y, rowp, colp, wp, EP, NP)
    out = _final_call(accp, y, dis_col, b[None, :], R)
    return out[:N]


# speedup vs baseline: 1.0697x; 1.0327x over previous
import functools

import jax
import jax.numpy as jnp
from jax import lax
from jax.experimental import pallas as pl
from jax.experimental.pallas import tpu as pltpu
from jax.experimental.pallas import tpu_sc as plsc

NC, NS, L = 2, 16, 16
NW = NC * NS
CH = 64
SPLIT0_PCT = 70

_SC_PARAMS = pltpu.CompilerParams(needs_layout_passes=False)


def _mesh():
    return plsc.VectorSubcoreMesh(core_axis_name="c", subcore_axis_name="s")


@functools.partial(jax.jit, static_argnums=(2, 3))
def _dis_call(colp, wp, EP, NP):
    EPW = EP // NS
    S = NP // NS

    @functools.partial(
        pl.kernel,
        out_type=jax.ShapeDtypeStruct((NP,), jnp.float32),
        mesh=_mesh(),
        compiler_params=_SC_PARAMS,
        scratch_types=[
            pltpu.VMEM_SHARED((NS, NP), jnp.float32),
            pltpu.VMEM((EPW,), jnp.int32),
            pltpu.VMEM((EPW,), jnp.float32),
            pltpu.VMEM((NP,), jnp.float32),
            pltpu.VMEM((S,), jnp.float32),
            pltpu.VMEM((S,), jnp.float32),
        ],
    )
    def dis_kernel(col_hbm, w_hbm, dis_hbm, part_sh, col_v, w_v, deg_v, racc_v, rtmp_v):
        cid = lax.axis_index("c")
        sid = lax.axis_index("s")

        def z(i, _):
            deg_v[pl.ds(i * L, L)] = jnp.zeros((L,), jnp.float32)
            return 0

        lax.fori_loop(0, NP // L, z, 0)
        base = sid * EPW
        pltpu.sync_copy(col_hbm.at[pl.ds(base, EPW)], col_v)
        pltpu.sync_copy(w_hbm.at[pl.ds(base, EPW)], w_v)

        def body(i, _):
            idx = col_v[pl.ds(i * L, L)]
            wv = w_v[pl.ds(i * L, L)]
            plsc.addupdate_scatter(deg_v, [idx], wv)
            return 0

        lax.fori_loop(0, EPW // L, body, 0)
        pltpu.sync_copy(deg_v, part_sh.at[sid])
        plsc.subcore_barrier()

        def z2(i, _):
            racc_v[pl.ds(i * L, L)] = jnp.ones((L,), jnp.float32)
            return 0

        lax.fori_loop(0, S // L, z2, 0)

        def red(t, _):
            pltpu.sync_copy(part_sh.at[t, pl.ds(sid * S, S)], rtmp_v)

            def addv(i, _):
                racc_v[pl.ds(i * L, L)] = (
                    racc_v[pl.ds(i * L, L)] + rtmp_v[pl.ds(i * L, L)]
                )
                return 0

            lax.fori_loop(0, S // L, addv, 0)
            return 0

        lax.fori_loop(0, NS, red, 0)

        def nrs(i, _):
            xx = racc_v[pl.ds(i * L, L)]
            ii = plsc.bitcast(xx, jnp.int32)
            ii = 0x5F3759DF - (ii >> 1)
            yy = plsc.bitcast(ii, jnp.float32)
            half = xx * 0.5
            yy = yy * (1.5 - half * yy * yy)
            yy = yy * (1.5 - half * yy * yy)
            yy = yy * (1.5 - half * yy * yy)
            racc_v[pl.ds(i * L, L)] = yy
            return 0

        lax.fori_loop(0, S // L, nrs, 0)

        @pl.when(cid == 0)
        def _():
            pltpu.sync_copy(racc_v, dis_hbm.at[pl.ds(sid * S, S)])

    return dis_kernel(colp, wp)


def _y_call(xp, W, R):
    NP = xp.shape[0]
    G = NP // R
    D = W.shape[1]

    def body(x_ref, w_ref, y_ref, sinv_ref):
        @pl.when(pl.program_id(0) == 0)
        def _():
            Wm = w_ref[...]
            A = lax.dot_general(
                Wm, Wm, (((1,), (1,)), ((), ())), preferred_element_type=jnp.float32
            )
            r_i = lax.broadcasted_iota(jnp.int32, (8, D), 0).astype(jnp.float32)
            c_i = lax.broadcasted_iota(jnp.int32, (8, D), 1).astype(jnp.float32)
            t = (r_i * 128.0 + c_i) * 0.6180339887
            v0 = t - jnp.floor(t) - 0.5

            def it(_, v):
                v = lax.dot_general(
                    v, A, (((1,), (0,)), ((), ())),
                    preferred_element_type=jnp.float32,
                )
                nrm = lax.rsqrt(jnp.sum(v * v, axis=1, keepdims=True) + 1e-30)
                return v * nrm

            v = lax.fori_loop(0, 80, it, v0)
            av = lax.dot_general(
                v, A, (((1,), (0,)), ((), ())), preferred_element_type=jnp.float32
            )
            rq = jnp.sum(av * v, axis=1, keepdims=True)
            lam = jnp.max(rq)
            sinv_ref[0, 0] = lax.rsqrt(lam)

        si = sinv_ref[0, 0]
        xw = lax.dot_general(
            x_ref[...], w_ref[...], (((1,), (1,)), ((), ())),
            preferred_element_type=jnp.float32,
        )
        y_ref[...] = xw * si

    return pl.pallas_call(
        body,
        grid=(G,),
        in_specs=[
            pl.BlockSpec((R, D), lambda i: (i, 0)),
            pl.BlockSpec((D, D), lambda i: (0, 0)),
        ],
        out_specs=pl.BlockSpec((R, D), lambda i: (i, 0)),
        out_shape=jax.ShapeDtypeStruct((NP, D), jnp.float32),
        scratch_shapes=[pltpu.SMEM((1, 1), jnp.float32)],
    )(xp, W)


@functools.partial(jax.jit, static_argnums=(5, 6))
def _scatter_call(dis, y, rowp, colp, wp, EP, NP):
    D = y.shape[1]
    NCHK_ALL = EP // CH // NS
    K0 = (NCHK_ALL * SPLIT0_PCT // 100) // 4 * 4
    K1 = NCHK_ALL - K0
    S = NP // NS
    NB = 4

    @functools.partial(
        pl.kernel,
        out_type=jax.ShapeDtypeStruct((NC, NP, D), jnp.float32),
        mesh=_mesh(),
        compiler_params=_SC_PARAMS,
        scratch_types=[
            pltpu.VMEM_SHARED((NP, D), jnp.float32),
            pltpu.VMEM((NB, CH), jnp.int32),
            pltpu.VMEM((NB, CH), jnp.int32),
            pltpu.VMEM((NB, CH), jnp.float32),
            pltpu.VMEM((NB, CH, D), jnp.float32),
            pltpu.VMEM((32, D), jnp.float32),
            pltpu.VMEM((NP,), jnp.float32),
            pltpu.SemaphoreType.DMA,
            pltpu.SemaphoreType.DMA,
            pltpu.SemaphoreType.DMA,
            pltpu.SemaphoreType.DMA,
            pltpu.SemaphoreType.DMA,
            pltpu.SemaphoreType.DMA,
            pltpu.SemaphoreType.DMA,
            pltpu.SemaphoreType.DMA,
        ],
    )
    def scat(dis_hbm, y_hbm, row_hbm, col_hbm, w_hbm, out_hbm,
             acc_sh, ri_v, ci_v, w_v, rows_v, zb_v, dis_v,
             is0, is1, is2, is3, gs0, gs1, gs2, gs3):
        cid = lax.axis_index("c")
        sid = lax.axis_index("s")
        nchk = jnp.where(cid == 0, K0, K1)
        base_chunk = jnp.where(cid == 0, sid * K0, NS * K0 + sid * K1)
        base = base_chunk * CH
        isem = (is0, is1, is2, is3)
        gsem = (gs0, gs1, gs2, gs3)

        def start_idx(c, b):
            off = base + c * CH
            pltpu.async_copy(row_hbm.at[pl.ds(off, CH)], ri_v.at[b], isem[b])
            pltpu.async_copy(col_hbm.at[pl.ds(off, CH)], ci_v.at[b], isem[b])
            pltpu.async_copy(w_hbm.at[pl.ds(off, CH)], w_v.at[b], isem[b])

        def wait_idx(c, b):
            off = base + c * CH
            pltpu.make_async_copy(row_hbm.at[pl.ds(off, CH)], ri_v.at[b], isem[b]).wait()
            pltpu.make_async_copy(col_hbm.at[pl.ds(off, CH)], ci_v.at[b], isem[b]).wait()
            pltpu.make_async_copy(w_hbm.at[pl.ds(off, CH)], w_v.at[b], isem[b]).wait()

        def start_g(b):
            pltpu.async_copy(y_hbm.at[ri_v.at[b]], rows_v.at[b], gsem[b])

        def wait_g(b):
            pltpu.make_async_copy(y_hbm.at[ri_v.at[b]], rows_v.at[b], gsem[b]).wait()

        def process(b):
            for q in range(CH // L):
                rr = ri_v[b, pl.ds(q * L, L)]
                cc = ci_v[b, pl.ds(q * L, L)]
                dr = plsc.load_gather(dis_v, [rr])
                dc = plsc.load_gather(dis_v, [cc])
                w_v[b, pl.ds(q * L, L)] = w_v[b, pl.ds(q * L, L)] * dr * dc

            def scale(r, _):
                wr = plsc.load_gather(w_v.at[b], [jnp.full((L,), r, jnp.int32)])
                for j in range(D // L):
                    rows_v[b, r, pl.ds(j * L, L)] = rows_v[b, r, pl.ds(j * L, L)] * wr
                return 0

            lax.fori_loop(0, CH, scale, 0)
            pltpu.sync_copy(rows_v.at[b], acc_sh.at[ci_v.at[b]], add=True)

        for b in range(NB):
            start_idx(b, b)
        pltpu.sync_copy(dis_hbm, dis_v)

        def zr(i, _):
            for j in range(D // L):
                zb_v[i, pl.ds(j * L, L)] = jnp.zeros((L,), jnp.float32)
            return 0

        lax.fori_loop(0, 32, zr, 0)

        def zs(i, _):
            pltpu.sync_copy(zb_v, acc_sh.at[pl.ds(sid * S + i * 32, 32)])
            return 0

        lax.fori_loop(0, S // 32, zs, 0)

        wait_idx(0, 0)
        start_g(0)
        wait_idx(1, 1)
        start_g(1)
        plsc.subcore_barrier()

        def body(g, _):
            c0 = 4 * g
            for b in range(NB):
                cc = c0 + b

                @pl.when(cc + 2 < nchk)
                def _():
                    wait_idx(cc + 2, (b + 2) % NB)
                    start_g((b + 2) % NB)

                wait_g(b)
                process(b)

                @pl.when(cc + 4 < nchk)
                def _():
                    start_idx(cc + 4, b)

            return 0

        lax.fori_loop(0, nchk // 4, body, 0)
        plsc.subcore_barrier()
        pltpu.sync_copy(acc_sh.at[pl.ds(sid * S, S)], out_hbm.at[cid, pl.ds(sid * S, S)])

    return scat(dis, y, rowp, colp, wp)


def _final_call(accp, y, dis_col, b2, R):
    NP, D = y.shape
    G = NP // R

    def body(a_ref, y_ref, dis_ref, b_ref, o_ref):
        d = dis_ref[...]
        o_ref[...] = a_ref[0] + a_ref[1] + y_ref[...] * (d * d) + b_ref[...]

    return pl.pallas_call(
        body,
        grid=(G,),
        in_specs=[
            pl.BlockSpec((NC, R, D), lambda i: (0, i, 0)),
            pl.BlockSpec((R, D), lambda i: (i, 0)),
            pl.BlockSpec((R, 1), lambda i: (i, 0)),
            pl.BlockSpec((1, D), lambda i: (0, 0)),
        ],
        out_specs=pl.BlockSpec((R, D), lambda i: (i, 0)),
        out_shape=jax.ShapeDtypeStruct((NP, D), jnp.float32),
    )(accp, y, dis_col, b2)


def kernel(x, edge_index, edge_weight, W, b):
    N, _ = x.shape
    E = edge_weight.shape[0]
    D = W.shape[1]

    NP = ((N + 1 + 2047) // 2048) * 2048
    EP = ((E + 4 * NW * CH - 1) // (4 * NW * CH)) * (4 * NW * CH)
    R = 2048

    row = edge_index[0]
    col = edge_index[1]
    pad_e = EP - E
    rowp = jnp.concatenate([row, jnp.zeros((pad_e,), jnp.int32)])
    colp = jnp.concatenate([col, jnp.full((pad_e,), N, jnp.int32)])
    wp = jnp.concatenate([edge_weight, jnp.zeros((pad_e,), jnp.float32)])
    xp = jnp.concatenate([x, jnp.zeros((NP - N, D), jnp.float32)], axis=0)

    dis = _dis_call(colp, wp, EP, NP)
    dis_col = dis[:, None]
    y = _y_call(xp, W, R)
    accp = _scatter_call(dis, y, rowp, colp, wp, EP, NP)
    out = _final_call(accp, y, dis_col, b[None, :], R)
    return out[:N]


# speedup vs baseline: 1.0731x; 1.0031x over previous
import functools

import jax
import jax.numpy as jnp
from jax import lax
from jax.experimental import pallas as pl
from jax.experimental.pallas import tpu as pltpu
from jax.experimental.pallas import tpu_sc as plsc

NC, NS, L = 2, 16, 16
NW = NC * NS
CH = 64
SPLIT0_PCT = 74

_SC_PARAMS = pltpu.CompilerParams(needs_layout_passes=False)


def _mesh():
    return plsc.VectorSubcoreMesh(core_axis_name="c", subcore_axis_name="s")


@functools.partial(jax.jit, static_argnums=(2, 3))
def _dis_call(colp, wp, EP, NP):
    EPW = EP // NS
    S = NP // NS

    @functools.partial(
        pl.kernel,
        out_type=jax.ShapeDtypeStruct((NP,), jnp.float32),
        mesh=_mesh(),
        compiler_params=_SC_PARAMS,
        scratch_types=[
            pltpu.VMEM_SHARED((NS, NP), jnp.float32),
            pltpu.VMEM((EPW,), jnp.int32),
            pltpu.VMEM((EPW,), jnp.float32),
            pltpu.VMEM((NP,), jnp.float32),
            pltpu.VMEM((S,), jnp.float32),
            pltpu.VMEM((S,), jnp.float32),
        ],
    )
    def dis_kernel(col_hbm, w_hbm, dis_hbm, part_sh, col_v, w_v, deg_v, racc_v, rtmp_v):
        cid = lax.axis_index("c")
        sid = lax.axis_index("s")

        def z(i, _):
            deg_v[pl.ds(i * L, L)] = jnp.zeros((L,), jnp.float32)
            return 0

        lax.fori_loop(0, NP // L, z, 0)
        base = sid * EPW
        pltpu.sync_copy(col_hbm.at[pl.ds(base, EPW)], col_v)
        pltpu.sync_copy(w_hbm.at[pl.ds(base, EPW)], w_v)

        def body(i, _):
            idx = col_v[pl.ds(i * L, L)]
            wv = w_v[pl.ds(i * L, L)]
            plsc.addupdate_scatter(deg_v, [idx], wv)
            return 0

        lax.fori_loop(0, EPW // L, body, 0)
        pltpu.sync_copy(deg_v, part_sh.at[sid])
        plsc.subcore_barrier()

        def z2(i, _):
            racc_v[pl.ds(i * L, L)] = jnp.ones((L,), jnp.float32)
            return 0

        lax.fori_loop(0, S // L, z2, 0)

        def red(t, _):
            pltpu.sync_copy(part_sh.at[t, pl.ds(sid * S, S)], rtmp_v)

            def addv(i, _):
                racc_v[pl.ds(i * L, L)] = (
                    racc_v[pl.ds(i * L, L)] + rtmp_v[pl.ds(i * L, L)]
                )
                return 0

            lax.fori_loop(0, S // L, addv, 0)
            return 0

        lax.fori_loop(0, NS, red, 0)

        def nrs(i, _):
            xx = racc_v[pl.ds(i * L, L)]
            ii = plsc.bitcast(xx, jnp.int32)
            ii = 0x5F3759DF - (ii >> 1)
            yy = plsc.bitcast(ii, jnp.float32)
            half = xx * 0.5
            yy = yy * (1.5 - half * yy * yy)
            yy = yy * (1.5 - half * yy * yy)
            yy = yy * (1.5 - half * yy * yy)
            racc_v[pl.ds(i * L, L)] = yy
            return 0

        lax.fori_loop(0, S // L, nrs, 0)

        @pl.when(cid == 0)
        def _():
            pltpu.sync_copy(racc_v, dis_hbm.at[pl.ds(sid * S, S)])

    return dis_kernel(colp, wp)


def _y_call(xp, W, R):
    NP = xp.shape[0]
    G = NP // R
    D = W.shape[1]

    def body(x_ref, w_ref, y_ref, sinv_ref):
        @pl.when(pl.program_id(0) == 0)
        def _():
            Wm = w_ref[...]
            A = lax.dot_general(
                Wm, Wm, (((1,), (1,)), ((), ())), preferred_element_type=jnp.float32
            )
            r_i = lax.broadcasted_iota(jnp.int32, (8, D), 0).astype(jnp.float32)
            c_i = lax.broadcasted_iota(jnp.int32, (8, D), 1).astype(jnp.float32)
            t = (r_i * 128.0 + c_i) * 0.6180339887
            v0 = t - jnp.floor(t) - 0.5

            def it(_, v):
                v = lax.dot_general(
                    v, A, (((1,), (0,)), ((), ())),
                    preferred_element_type=jnp.float32,
                )
                nrm = lax.rsqrt(jnp.sum(v * v, axis=1, keepdims=True) + 1e-30)
                return v * nrm

            v = lax.fori_loop(0, 80, it, v0)
            av = lax.dot_general(
                v, A, (((1,), (0,)), ((), ())), preferred_element_type=jnp.float32
            )
            rq = jnp.sum(av * v, axis=1, keepdims=True)
            lam = jnp.max(rq)
            sinv_ref[0, 0] = lax.rsqrt(lam)

        si = sinv_ref[0, 0]
        xw = lax.dot_general(
            x_ref[...], w_ref[...], (((1,), (1,)), ((), ())),
            preferred_element_type=jnp.float32,
        )
        y_ref[...] = xw * si

    return pl.pallas_call(
        body,
        grid=(G,),
        in_specs=[
            pl.BlockSpec((R, D), lambda i: (i, 0)),
            pl.BlockSpec((D, D), lambda i: (0, 0)),
        ],
        out_specs=pl.BlockSpec((R, D), lambda i: (i, 0)),
        out_shape=jax.ShapeDtypeStruct((NP, D), jnp.float32),
        scratch_shapes=[pltpu.SMEM((1, 1), jnp.float32)],
    )(xp, W)


@functools.partial(jax.jit, static_argnums=(5, 6))
def _scatter_call(dis, y, rowp, colp, wp, EP, NP):
    D = y.shape[1]
    NCHK_ALL = EP // CH // NS
    K0 = (NCHK_ALL * SPLIT0_PCT // 100) // 4 * 4
    K1 = NCHK_ALL - K0
    S = NP // NS
    NB = 4

    @functools.partial(
        pl.kernel,
        out_type=jax.ShapeDtypeStruct((NC, NP, D), jnp.float32),
        mesh=_mesh(),
        compiler_params=_SC_PARAMS,
        scratch_types=[
            pltpu.VMEM_SHARED((NP, D), jnp.float32),
            pltpu.VMEM((NB, CH), jnp.int32),
            pltpu.VMEM((NB, CH), jnp.int32),
            pltpu.VMEM((NB, CH), jnp.float32),
            pltpu.VMEM((NB, CH, D), jnp.float32),
            pltpu.VMEM((32, D), jnp.float32),
            pltpu.VMEM((NP,), jnp.float32),
            pltpu.SemaphoreType.DMA,
            pltpu.SemaphoreType.DMA,
            pltpu.SemaphoreType.DMA,
            pltpu.SemaphoreType.DMA,
            pltpu.SemaphoreType.DMA,
            pltpu.SemaphoreType.DMA,
            pltpu.SemaphoreType.DMA,
            pltpu.SemaphoreType.DMA,
        ],
    )
    def scat(dis_hbm, y_hbm, row_hbm, col_hbm, w_hbm, out_hbm,
             acc_sh, ri_v, ci_v, w_v, rows_v, zb_v, dis_v,
             is0, is1, is2, is3, gs0, gs1, gs2, gs3):
        cid = lax.axis_index("c")
        sid = lax.axis_index("s")
        nchk = jnp.where(cid == 0, K0, K1)
        base_chunk = jnp.where(cid == 0, sid * K0, NS * K0 + sid * K1)
        base = base_chunk * CH
        isem = (is0, is1, is2, is3)
        gsem = (gs0, gs1, gs2, gs3)

        def start_idx(c, b):
            off = base + c * CH
            pltpu.async_copy(row_hbm.at[pl.ds(off, CH)], ri_v.at[b], isem[b])
            pltpu.async_copy(col_hbm.at[pl.ds(off, CH)], ci_v.at[b], isem[b])
            pltpu.async_copy(w_hbm.at[pl.ds(off, CH)], w_v.at[b], isem[b])

        def wait_idx(c, b):
            off = base + c * CH
            pltpu.make_async_copy(row_hbm.at[pl.ds(off, CH)], ri_v.at[b], isem[b]).wait()
            pltpu.make_async_copy(col_hbm.at[pl.ds(off, CH)], ci_v.at[b], isem[b]).wait()
            pltpu.make_async_copy(w_hbm.at[pl.ds(off, CH)], w_v.at[b], isem[b]).wait()

        def start_g(b):
            pltpu.async_copy(y_hbm.at[ri_v.at[b]], rows_v.at[b], gsem[b])

        def wait_g(b):
            pltpu.make_async_copy(y_hbm.at[ri_v.at[b]], rows_v.at[b], gsem[b]).wait()

        def process(b):
            for q in range(CH // L):
                rr = ri_v[b, pl.ds(q * L, L)]
                cc = ci_v[b, pl.ds(q * L, L)]
                dr = plsc.load_gather(dis_v, [rr])
                dc = plsc.load_gather(dis_v, [cc])
                w_v[b, pl.ds(q * L, L)] = w_v[b, pl.ds(q * L, L)] * dr * dc

            def scale(r, _):
                wr = plsc.load_gather(w_v.at[b], [jnp.full((L,), r, jnp.int32)])
                for j in range(D // L):
                    rows_v[b, r, pl.ds(j * L, L)] = rows_v[b, r, pl.ds(j * L, L)] * wr
                return 0

            lax.fori_loop(0, CH, scale, 0)
            pltpu.sync_copy(rows_v.at[b], acc_sh.at[ci_v.at[b]], add=True)

        for b in range(NB):
            start_idx(b, b)
        pltpu.sync_copy(dis_hbm, dis_v)

        def zr(i, _):
            for j in range(D // L):
                zb_v[i, pl.ds(j * L, L)] = jnp.zeros((L,), jnp.float32)
            return 0

        lax.fori_loop(0, 32, zr, 0)

        def zs(i, _):
            pltpu.sync_copy(zb_v, acc_sh.at[pl.ds(sid * S + i * 32, 32)])
            return 0

        lax.fori_loop(0, S // 32, zs, 0)

        wait_idx(0, 0)
        start_g(0)
        wait_idx(1, 1)
        start_g(1)
        plsc.subcore_barrier()

        def body(g, _):
            c0 = 4 * g
            for b in range(NB):
                cc = c0 + b

                @pl.when(cc + 2 < nchk)
                def _():
                    wait_idx(cc + 2, (b + 2) % NB)
                    start_g((b + 2) % NB)

                wait_g(b)
                process(b)

                @pl.when(cc + 4 < nchk)
                def _():
                    start_idx(cc + 4, b)

            return 0

        lax.fori_loop(0, nchk // 4, body, 0)
        plsc.subcore_barrier()
        pltpu.sync_copy(acc_sh.at[pl.ds(sid * S, S)], out_hbm.at[cid, pl.ds(sid * S, S)])

    return scat(dis, y, rowp, colp, wp)


def _final_call(accp, y, dis_col, b2, R):
    NP, D = y.shape
    G = NP // R

    def body(a_ref, y_ref, dis_ref, b_ref, o_ref):
        d = dis_ref[...]
        o_ref[...] = a_ref[0] + a_ref[1] + y_ref[...] * (d * d) + b_ref[...]

    return pl.pallas_call(
        body,
        grid=(G,),
        in_specs=[
            pl.BlockSpec((NC, R, D), lambda i: (0, i, 0)),
            pl.BlockSpec((R, D), lambda i: (i, 0)),
            pl.BlockSpec((R, 1), lambda i: (i, 0)),
            pl.BlockSpec((1, D), lambda i: (0, 0)),
        ],
        out_specs=pl.BlockSpec((R, D), lambda i: (i, 0)),
        out_shape=jax.ShapeDtypeStruct((NP, D), jnp.float32),
    )(accp, y, dis_col, b2)


def kernel(x, edge_index, edge_weight, W, b):
    N, _ = x.shape
    E = edge_weight.shape[0]
    D = W.shape[1]

    NP = ((N + 1 + 2047) // 2048) * 2048
    EP = ((E + 4 * NW * CH - 1) // (4 * NW * CH)) * (4 * NW * CH)
    R = 2048

    row = edge_index[0]
    col = edge_index[1]
    pad_e = EP - E
    rowp = jnp.concatenate([row, jnp.zeros((pad_e,), jnp.int32)])
    colp = jnp.concatenate([col, jnp.full((pad_e,), N, jnp.int32)])
    wp = jnp.concatenate([edge_weight, jnp.zeros((pad_e,), jnp.float32)])
    xp = jnp.concatenate([x, jnp.zeros((NP - N, D), jnp.float32)], axis=0)

    dis = _dis_call(colp, wp, EP, NP)
    dis_col = dis[:, None]
    y = _y_call(xp, W, R)
    accp = _scatter_call(dis, y, rowp, colp, wp, EP, NP)
    out = _final_call(accp, y, dis_col, b[None, :], R)
    return out[:N]
